# Initial kernel scaffold; baseline (speedup 1.0000x reference)
import jax
import jax.numpy as jnp
from jax.experimental import pallas as pl


def kernel(x, img_pat):
    raise NotImplementedError("write your pallas kernel here")


# speedup vs baseline: 1.0743x; 1.0743x over previous
import functools

import jax
import jax.numpy as jnp
from jax import lax
from jax.experimental import pallas as pl
from jax.experimental.pallas import tpu as pltpu
from jax.experimental.pallas import tpu_sc as plsc

N, L, D = 64, 1024, 768
NUM_BINS = 10
LEN_KEEP = 256


def _score_body(img_ref, idr_ref, mask_ref, keep_ref):
    n = pl.program_id(0)
    t = img_ref[0]

    mn = jnp.min(t, axis=-1, keepdims=True)
    mx = jnp.max(t, axis=-1, keepdims=True)
    norm = (t - mn) / (mx - mn + 1e-19)
    q = jnp.floor(norm * (NUM_BINS - 1)).astype(jnp.int32)
    q = jnp.clip(q, 0, NUM_BINS - 1)
    counts = [
        jnp.sum((q == b).astype(jnp.float32), axis=-1, keepdims=True)
        for b in range(NUM_BINS)
    ]
    total = counts[0]
    for b in range(1, NUM_BINS):
        total = total + counts[b]
    ent = None
    for b in range(NUM_BINS):
        p = counts[b] / total
        term = p * jnp.log(p + 1e-09)
        ent = term if ent is None else ent + term
    e_col = -ent

    e_row = jnp.reshape(e_col, (1, L))
    j_idx = lax.broadcasted_iota(jnp.int32, (1, L), 1)
    rank = jnp.zeros((1, L), jnp.int32)
    CH = 128
    for c in range(L // CH):
        ei = e_col[c * CH:(c + 1) * CH, :]
        ii = c * CH + lax.broadcasted_iota(jnp.int32, (CH, 1), 0)
        beats = (ei > e_row) | ((ei == e_row) & (ii < j_idx))
        rank = rank + jnp.sum(beats.astype(jnp.int32), axis=0, keepdims=True)

    idr_ref[0] = rank
    mask_ref[0] = (rank >= LEN_KEEP).astype(jnp.float32)

    r_idx = lax.broadcasted_iota(jnp.int32, (LEN_KEEP, 1), 0)
    oh = (rank == r_idx)
    vals = jnp.where(oh, jnp.broadcast_to(j_idx, (LEN_KEEP, L)), 0)
    ids_local = jnp.sum(vals, axis=1, keepdims=True)
    keep_ref[0] = jnp.reshape(ids_local, (1, LEN_KEEP)) + n * L


def _score(img_pat):
    return pl.pallas_call(
        _score_body,
        grid=(N,),
        in_specs=[pl.BlockSpec((1, L, D), lambda n: (n, 0, 0))],
        out_specs=[
            pl.BlockSpec((1, 1, L), lambda n: (n, 0, 0)),
            pl.BlockSpec((1, 1, L), lambda n: (n, 0, 0)),
            pl.BlockSpec((1, 1, LEN_KEEP), lambda n: (n, 0, 0)),
        ],
        out_shape=[
            jax.ShapeDtypeStruct((N, 1, L), jnp.int32),
            jax.ShapeDtypeStruct((N, 1, L), jnp.float32),
            jax.ShapeDtypeStruct((N, 1, LEN_KEEP), jnp.int32),
        ],
    )(img_pat)


_B = N * LEN_KEEP
_NW = 32
_BPW = _B // _NW
_CHUNK = 128


def _gather_body(x_hbm, ids_hbm, out_hbm, idx_v, rows_v, sem):
    wid = lax.axis_index("s") * 2 + lax.axis_index("c")
    base = wid * _BPW
    pltpu.sync_copy(ids_hbm.at[pl.ds(base, _BPW)], idx_v)
    for c in range(_BPW // _CHUNK):
        pltpu.async_copy(
            x_hbm.at[idx_v.at[pl.ds(c * _CHUNK, _CHUNK)]], rows_v, sem
        ).wait()
        pltpu.sync_copy(rows_v, out_hbm.at[pl.ds(base + c * _CHUNK, _CHUNK)])


@functools.cache
def _make_gather():
    return pl.kernel(
        _gather_body,
        out_type=jax.ShapeDtypeStruct((_B, D), jnp.float32),
        mesh=plsc.VectorSubcoreMesh(core_axis_name="c", subcore_axis_name="s"),
        scratch_types=[
            pltpu.VMEM((_BPW,), jnp.int32),
            pltpu.VMEM((_CHUNK, D), jnp.float32),
            pltpu.SemaphoreType.DMA,
        ],
    )


def kernel(x, img_pat):
    ids_restore3, mask3, keep3 = _score(img_pat)
    ids = keep3.reshape(_B)
    x_masked = _make_gather()(x.reshape(N * L, D), ids).reshape(N, LEN_KEEP, D)
    return (x_masked, mask3.reshape(N, L), ids_restore3.reshape(N, L))


# speedup vs baseline: 1.1975x; 1.1147x over previous
import functools

import jax
import jax.numpy as jnp
from jax import lax
from jax.experimental import pallas as pl
from jax.experimental.pallas import tpu as pltpu
from jax.experimental.pallas import tpu_sc as plsc

N, L, D = 64, 1024, 768
NUM_BINS = 10
LEN_KEEP = 256


def _score_body(img_ref, idr_ref, mask_ref, keep_ref):
    n = pl.program_id(0)
    t = img_ref[0]

    mn = jnp.min(t, axis=-1, keepdims=True)
    mx = jnp.max(t, axis=-1, keepdims=True)
    norm = (t - mn) / (mx - mn + 1e-19)
    q = jnp.clip(jnp.floor(norm * (NUM_BINS - 1)), 0.0, NUM_BINS - 1.0)
    counts = [
        jnp.sum((q == float(b)).astype(jnp.float32), axis=-1)
        for b in range(NUM_BINS - 1)
    ]
    rest = counts[0]
    for b in range(1, NUM_BINS - 1):
        rest = rest + counts[b]
    counts.append(float(D) - rest)
    total = counts[0]
    for b in range(1, NUM_BINS):
        total = total + counts[b]
    ent = None
    for b in range(NUM_BINS):
        p = counts[b] / total
        term = p * jnp.log(p + 1e-09)
        ent = term if ent is None else ent + term
    e = -ent

    e_col = jnp.reshape(e, (L, 1))
    e_row = jnp.reshape(e, (1, L))
    j_idx = lax.broadcasted_iota(jnp.int32, (1, L), 1)
    rank = jnp.zeros((1, L), jnp.int32)
    CH = 128
    for c in range(L // CH):
        ei = e_col[c * CH:(c + 1) * CH, :]
        ii = c * CH + lax.broadcasted_iota(jnp.int32, (CH, 1), 0)
        beats = (ei > e_row) | ((ei == e_row) & (ii < j_idx))
        rank = rank + jnp.sum(beats.astype(jnp.int32), axis=0, keepdims=True)

    idr_ref[0] = rank
    mask_ref[0] = (rank >= LEN_KEEP).astype(jnp.float32)

    r_idx = lax.broadcasted_iota(jnp.int32, (LEN_KEEP, 1), 0)
    oh = (rank == r_idx)
    vals = jnp.where(oh, jnp.broadcast_to(j_idx, (LEN_KEEP, L)), 0)
    ids_local = jnp.sum(vals, axis=1)
    keep_ref[0] = jnp.reshape(ids_local, (1, LEN_KEEP)) + n * L


def _score(img_pat):
    return pl.pallas_call(
        _score_body,
        grid=(N,),
        in_specs=[pl.BlockSpec((1, L, D), lambda n: (n, 0, 0))],
        out_specs=[
            pl.BlockSpec((1, 1, L), lambda n: (n, 0, 0)),
            pl.BlockSpec((1, 1, L), lambda n: (n, 0, 0)),
            pl.BlockSpec((1, 1, LEN_KEEP), lambda n: (n, 0, 0)),
        ],
        out_shape=[
            jax.ShapeDtypeStruct((N, 1, L), jnp.int32),
            jax.ShapeDtypeStruct((N, 1, L), jnp.float32),
            jax.ShapeDtypeStruct((N, 1, LEN_KEEP), jnp.int32),
        ],
    )(img_pat)


_B = N * LEN_KEEP
_NW = 32
_BPW = _B // _NW
_CHUNK = 128


def _gather_body(x_hbm, ids_hbm, out_hbm, idx_v, rows_v, sem):
    wid = lax.axis_index("s") * 2 + lax.axis_index("c")
    base = wid * _BPW
    pltpu.sync_copy(ids_hbm.at[pl.ds(base, _BPW)], idx_v)
    for c in range(_BPW // _CHUNK):
        pltpu.async_copy(
            x_hbm.at[idx_v.at[pl.ds(c * _CHUNK, _CHUNK)]], rows_v, sem
        ).wait()
        pltpu.sync_copy(rows_v, out_hbm.at[pl.ds(base + c * _CHUNK, _CHUNK)])


@functools.cache
def _make_gather():
    return pl.kernel(
        _gather_body,
        out_type=jax.ShapeDtypeStruct((_B, D), jnp.float32),
        mesh=plsc.VectorSubcoreMesh(core_axis_name="c", subcore_axis_name="s"),
        scratch_types=[
            pltpu.VMEM((_BPW,), jnp.int32),
            pltpu.VMEM((_CHUNK, D), jnp.float32),
            pltpu.SemaphoreType.DMA,
        ],
    )


def kernel(x, img_pat):
    ids_restore3, mask3, keep3 = _score(img_pat)
    ids = keep3.reshape(_B)
    x_masked = _make_gather()(x.reshape(N * L, D), ids).reshape(N, LEN_KEEP, D)
    return (x_masked, mask3.reshape(N, L), ids_restore3.reshape(N, L))


# speedup vs baseline: 1.5363x; 1.2829x over previous
import functools

import jax
import jax.numpy as jnp
from jax import lax
from jax.experimental import pallas as pl
from jax.experimental.pallas import tpu as pltpu
from jax.experimental.pallas import tpu_sc as plsc

N, L, D = 64, 1024, 768
NUM_BINS = 10
LEN_KEEP = 256


def _score_body(img_ref, w_ref, idr_ref, mask_ref, keep_ref):
    n = pl.program_id(0)
    t = img_ref[0]

    mn = jnp.min(t, axis=-1, keepdims=True)
    mx = jnp.max(t, axis=-1, keepdims=True)
    norm9 = ((t - mn) / (mx - mn + 1e-19)) * (NUM_BINS - 1)
    partials = []
    for b in range(1, NUM_BINS):
        m = (norm9 >= float(b)).astype(jnp.float32)
        acc = m[:, 0:128]
        for k in range(1, D // 128):
            acc = acc + m[:, k * 128:(k + 1) * 128]
        partials.append(acc.astype(jnp.bfloat16))
    pcat = jnp.concatenate(partials, axis=1)
    csum = jax.lax.dot_general(
        pcat, w_ref[...], (((1,), (0,)), ((), ())),
        preferred_element_type=jnp.float32)
    ct = csum.T
    s = [ct[b - 1:b, :] for b in range(1, NUM_BINS)]
    counts = [float(D) - s[0]]
    for b in range(1, NUM_BINS - 1):
        counts.append(s[b - 1] - s[b])
    counts.append(s[NUM_BINS - 2])
    total = counts[0]
    for b in range(1, NUM_BINS):
        total = total + counts[b]
    ent = None
    for b in range(NUM_BINS):
        p = counts[b] / total
        term = p * jnp.log(p + 1e-09)
        ent = term if ent is None else ent + term
    e_row = -ent

    e_col = jnp.reshape(e_row, (L, 1))
    j_idx = lax.broadcasted_iota(jnp.int32, (1, L), 1)
    rank = jnp.zeros((1, L), jnp.int32)
    CH = 128
    for c in range(L // CH):
        ei = e_col[c * CH:(c + 1) * CH, :]
        ii = c * CH + lax.broadcasted_iota(jnp.int32, (CH, 1), 0)
        beats = (ei > e_row) | ((ei == e_row) & (ii < j_idx))
        rank = rank + jnp.sum(beats.astype(jnp.int32), axis=0, keepdims=True)

    idr_ref[0] = rank
    mask_ref[0] = (rank >= LEN_KEEP).astype(jnp.float32)

    r_idx = lax.broadcasted_iota(jnp.int32, (LEN_KEEP, 1), 0)
    oh = (rank == r_idx)
    vals = jnp.where(oh, jnp.broadcast_to(j_idx, (LEN_KEEP, L)), 0)
    ids_local = jnp.sum(vals, axis=1)
    keep_ref[0] = jnp.reshape(ids_local, (1, LEN_KEEP)) + n * L


def _score(img_pat):
    import numpy as np
    w = np.zeros(((NUM_BINS - 1) * 128, 128), np.float32)
    for b in range(NUM_BINS - 1):
        w[b * 128:(b + 1) * 128, b] = 1.0
    w = jnp.asarray(w, jnp.bfloat16)
    return pl.pallas_call(
        _score_body,
        grid=(N,),
        in_specs=[
            pl.BlockSpec((1, L, D), lambda n: (n, 0, 0)),
            pl.BlockSpec(((NUM_BINS - 1) * 128, 128), lambda n: (0, 0)),
        ],
        out_specs=[
            pl.BlockSpec((1, 1, L), lambda n: (n, 0, 0)),
            pl.BlockSpec((1, 1, L), lambda n: (n, 0, 0)),
            pl.BlockSpec((1, 1, LEN_KEEP), lambda n: (n, 0, 0)),
        ],
        out_shape=[
            jax.ShapeDtypeStruct((N, 1, L), jnp.int32),
            jax.ShapeDtypeStruct((N, 1, L), jnp.float32),
            jax.ShapeDtypeStruct((N, 1, LEN_KEEP), jnp.int32),
        ],
    )(img_pat, w)


_B = N * LEN_KEEP
_NW = 32
_BPW = _B // _NW
_CHUNK = 128


def _gather_body(x_hbm, ids_hbm, out_hbm, idx_v, rows_v, sem):
    wid = lax.axis_index("s") * 2 + lax.axis_index("c")
    base = wid * _BPW
    pltpu.sync_copy(ids_hbm.at[pl.ds(base, _BPW)], idx_v)
    for c in range(_BPW // _CHUNK):
        pltpu.async_copy(
            x_hbm.at[idx_v.at[pl.ds(c * _CHUNK, _CHUNK)]], rows_v, sem
        ).wait()
        pltpu.sync_copy(rows_v, out_hbm.at[pl.ds(base + c * _CHUNK, _CHUNK)])


@functools.cache
def _make_gather():
    return pl.kernel(
        _gather_body,
        out_type=jax.ShapeDtypeStruct((_B, D), jnp.float32),
        mesh=plsc.VectorSubcoreMesh(core_axis_name="c", subcore_axis_name="s"),
        scratch_types=[
            pltpu.VMEM((_BPW,), jnp.int32),
            pltpu.VMEM((_CHUNK, D), jnp.float32),
            pltpu.SemaphoreType.DMA,
        ],
    )


def kernel(x, img_pat):
    ids_restore3, mask3, keep3 = _score(img_pat)
    ids = keep3.reshape(_B)
    x_masked = _make_gather()(x.reshape(N * L, D), ids).reshape(N, LEN_KEEP, D)
    return (x_masked, mask3.reshape(N, L), ids_restore3.reshape(N, L))


# speedup vs baseline: 1.5376x; 1.0008x over previous
import functools

import jax
import jax.numpy as jnp
from jax import lax
from jax.experimental import pallas as pl
from jax.experimental.pallas import tpu as pltpu
from jax.experimental.pallas import tpu_sc as plsc

N, L, D = 64, 1024, 768
NUM_BINS = 10
LEN_KEEP = 256


def _score_body(img_ref, w_ref, idr_ref, mask_ref, keep_ref):
    n = pl.program_id(0)
    t = img_ref[0]

    mn = jnp.min(t, axis=-1, keepdims=True)
    mx = jnp.max(t, axis=-1, keepdims=True)
    norm9 = ((t - mn) / (mx - mn + 1e-19)) * (NUM_BINS - 1)
    partials = []
    for b in range(1, NUM_BINS):
        m = (norm9 >= float(b)).astype(jnp.float32)
        acc = m[:, 0:128]
        for k in range(1, D // 128):
            acc = acc + m[:, k * 128:(k + 1) * 128]
        partials.append(acc.astype(jnp.bfloat16))
    pcat = jnp.concatenate(partials, axis=1)
    csum = jax.lax.dot_general(
        pcat, w_ref[...], (((1,), (0,)), ((), ())),
        preferred_element_type=jnp.float32)
    ct = csum.T
    s = [ct[b - 1:b, :] for b in range(1, NUM_BINS)]
    counts = [float(D) - s[0]]
    for b in range(1, NUM_BINS - 1):
        counts.append(s[b - 1] - s[b])
    counts.append(s[NUM_BINS - 2])
    total = counts[0]
    for b in range(1, NUM_BINS):
        total = total + counts[b]
    ent = None
    for b in range(NUM_BINS):
        p = counts[b] / total
        term = p * jnp.log(p + 1e-09)
        ent = term if ent is None else ent + term
    e_row = -ent

    e_col = jnp.reshape(e_row, (L, 1))
    j_idx = lax.broadcasted_iota(jnp.int32, (1, L), 1)
    rank = jnp.zeros((1, L), jnp.int32)
    CH = 128
    for c in range(L // CH):
        ei = e_col[c * CH:(c + 1) * CH, :]
        ii = c * CH + lax.broadcasted_iota(jnp.int32, (CH, 1), 0)
        beats = (ei > e_row) | ((ei == e_row) & (ii < j_idx))
        rank = rank + jnp.sum(beats.astype(jnp.int32), axis=0, keepdims=True)

    idr_ref[0] = rank
    mask_ref[0] = (rank >= LEN_KEEP).astype(jnp.float32)

    r_idx = lax.broadcasted_iota(jnp.int32, (LEN_KEEP, 1), 0)
    oh = (rank == r_idx)
    vals = jnp.where(oh, jnp.broadcast_to(j_idx, (LEN_KEEP, L)), 0)
    ids_local = jnp.sum(vals, axis=1)
    keep_ref[0] = jnp.reshape(ids_local, (1, LEN_KEEP)) + n * L


def _score(img_pat):
    import numpy as np
    w = np.zeros(((NUM_BINS - 1) * 128, 128), np.float32)
    for b in range(NUM_BINS - 1):
        w[b * 128:(b + 1) * 128, b] = 1.0
    w = jnp.asarray(w, jnp.bfloat16)
    return pl.pallas_call(
        _score_body,
        grid=(N,),
        in_specs=[
            pl.BlockSpec((1, L, D), lambda n: (n, 0, 0)),
            pl.BlockSpec(((NUM_BINS - 1) * 128, 128), lambda n: (0, 0)),
        ],
        out_specs=[
            pl.BlockSpec((1, 1, L), lambda n: (n, 0, 0)),
            pl.BlockSpec((1, 1, L), lambda n: (n, 0, 0)),
            pl.BlockSpec((1, 1, LEN_KEEP), lambda n: (n, 0, 0)),
        ],
        out_shape=[
            jax.ShapeDtypeStruct((N, 1, L), jnp.int32),
            jax.ShapeDtypeStruct((N, 1, L), jnp.float32),
            jax.ShapeDtypeStruct((N, 1, LEN_KEEP), jnp.int32),
        ],
    )(img_pat, w)


_B = N * LEN_KEEP
_NW = 32
_BPW = _B // _NW
_CHUNK = 64


def _gather_body(x_hbm, ids_hbm, out_hbm, idx_v, rows0, rows1, sem0, sem1):
    wid = lax.axis_index("s") * 2 + lax.axis_index("c")
    base = wid * _BPW
    pltpu.sync_copy(ids_hbm.at[pl.ds(base, _BPW)], idx_v)
    bufs, sems = (rows0, rows1), (sem0, sem1)
    nck = _BPW // _CHUNK
    handles = [None] * nck
    handles[0] = pltpu.async_copy(
        x_hbm.at[idx_v.at[pl.ds(0, _CHUNK)]], rows0, sem0)
    for c in range(nck):
        if c + 1 < nck:
            handles[c + 1] = pltpu.async_copy(
                x_hbm.at[idx_v.at[pl.ds((c + 1) * _CHUNK, _CHUNK)]],
                bufs[(c + 1) % 2], sems[(c + 1) % 2])
        handles[c].wait()
        pltpu.sync_copy(bufs[c % 2],
                        out_hbm.at[pl.ds(base + c * _CHUNK, _CHUNK)])


@functools.cache
def _make_gather():
    return pl.kernel(
        _gather_body,
        out_type=jax.ShapeDtypeStruct((_B, D), jnp.float32),
        mesh=plsc.VectorSubcoreMesh(core_axis_name="c", subcore_axis_name="s"),
        scratch_types=[
            pltpu.VMEM((_BPW,), jnp.int32),
            pltpu.VMEM((_CHUNK, D), jnp.float32),
            pltpu.VMEM((_CHUNK, D), jnp.float32),
            pltpu.SemaphoreType.DMA,
            pltpu.SemaphoreType.DMA,
        ],
    )


def kernel(x, img_pat):
    ids_restore3, mask3, keep3 = _score(img_pat)
    ids = keep3.reshape(_B)
    x_masked = _make_gather()(x.reshape(N * L, D), ids).reshape(N, LEN_KEEP, D)
    return (x_masked, mask3.reshape(N, L), ids_restore3.reshape(N, L))
